# baseline (speedup 1.0000x reference)
import functools

import jax
import jax.numpy as jnp
from jax import lax
from jax.experimental import pallas as pl
from jax.experimental.pallas import tpu as pltpu
from jax.experimental.pallas import tpu_sc as plsc

R = 128
N = 32768
L = 16
NC = 2
NS = 16
NW = NC * NS
SC_ROWS = 32
TC_ROWS = R - SC_ROWS
RPW = SC_ROWS // NW
UNROLL = 4
STEPS = N // (L * UNROLL)

_mesh = plsc.VectorSubcoreMesh(core_axis_name="c", subcore_axis_name="s")


@functools.partial(
    pl.kernel,
    mesh=_mesh,
    out_type=jax.ShapeDtypeStruct((NW, L), jnp.float32),
    scratch_types=[
        pltpu.VMEM((N,), jnp.float32),
        pltpu.VMEM((8, 128), jnp.float32),
        pltpu.VMEM((L,), jnp.float32),
        pltpu.SemaphoreType.DMA,
    ],
)
def _sc_argmax_gather(m_hbm, hypo_hbm, out_hbm, buf, gwin, obuf, sem):
    wid = lax.axis_index("s") * NC + lax.axis_index("c")
    row = TC_ROWS + wid
    lane = lax.iota(jnp.int32, L)

    pltpu.async_copy(m_hbm.at[row], buf, sem).wait()

    neg_inf = jnp.full((L,), -jnp.inf, jnp.float32)
    zeros = jnp.zeros((L,), jnp.int32)

    def body(i, carry):
        vmaxs, vits = carry
        base = i * (L * UNROLL)
        isplat = jnp.full((L,), i, jnp.int32)
        nmax, nit = [], []
        for u in range(UNROLL):
            v = buf[pl.ds(base + u * L, L)]
            mask = v > vmaxs[u]
            nmax.append(jnp.where(mask, v, vmaxs[u]))
            nit.append(jnp.where(mask, isplat, vits[u]))
        return tuple(nmax), tuple(nit)

    vmaxs, vits = lax.fori_loop(
        0, STEPS, body, ((neg_inf,) * UNROLL, (zeros,) * UNROLL))

    vmax = vmaxs[0]
    vidx = vits[0] * (L * UNROLL) + lane
    for u in range(1, UNROLL):
        vidx_u = vits[u] * (L * UNROLL) + (u * L) + lane
        take = (vmaxs[u] > vmax) | ((vmaxs[u] == vmax) & (vidx_u < vidx))
        vmax = jnp.where(take, vmaxs[u], vmax)
        vidx = jnp.where(take, vidx_u, vidx)

    for d in (8, 4, 2, 1):
        perm = jnp.bitwise_xor(lane, jnp.int32(d))
        vmax2 = vmax.at[perm].get(mode="promise_in_bounds")
        vidx2 = vidx.at[perm].get(mode="promise_in_bounds")
        take = (vmax2 > vmax) | ((vmax2 == vmax) & (vidx2 < vidx))
        vmax = jnp.where(take, vmax2, vmax)
        vidx = jnp.where(take, vidx2, vidx)

    col = vidx[0]
    row8 = pl.multiple_of(row & jnp.int32(-8), 8)
    col128 = pl.multiple_of(col & jnp.int32(-128), 128)
    pltpu.async_copy(
        hypo_hbm.at[pl.ds(row8, 8), pl.ds(col128, 128)], gwin, sem).wait()

    sub = row & jnp.int32(7)
    off = col & jnp.int32(127)
    v = gwin[sub, pl.ds(off & jnp.int32(-16), L)]
    obuf[...] = v.at[jnp.full((L,), off & jnp.int32(15))].get(
        mode="promise_in_bounds")
    pltpu.sync_copy(obuf, out_hbm.at[wid])


def _tc_body(m_ref, h_ref, o_ref):
    x = m_ref[...]
    h = h_ref[...]
    idx = lax.broadcasted_iota(jnp.int32, (8, N), 1)
    mx = jnp.max(x, axis=1, keepdims=True)
    eq = x == mx
    big = jnp.full((8, N), jnp.int32(2**30))
    am = jnp.min(jnp.where(eq, idx, big), axis=1)
    sel = idx == am[:, None]
    o_ref[...] = jnp.sum(jnp.where(sel, h, 0.0), axis=1).reshape(1, 8, 1)


_tc_argmax_gather = pl.pallas_call(
    _tc_body,
    grid=(TC_ROWS // 8,),
    in_specs=[
        pl.BlockSpec((8, N), lambda i: (i, 0)),
        pl.BlockSpec((8, N), lambda i: (i, 0)),
    ],
    out_specs=pl.BlockSpec((1, 8, 1), lambda i: (i, 0, 0)),
    out_shape=jax.ShapeDtypeStruct((TC_ROWS // 8, 8, 1), jnp.float32),
)


def kernel(hypo, m):
    out_sc = _sc_argmax_gather(m, hypo)
    out_tc = _tc_argmax_gather(m, hypo)
    return jnp.concatenate(
        [out_tc.reshape(TC_ROWS), out_sc[:, 0]])


# speedup vs baseline: 1.1382x; 1.1382x over previous
import functools

import jax
import jax.numpy as jnp
from jax import lax
from jax.experimental import pallas as pl
from jax.experimental.pallas import tpu as pltpu
from jax.experimental.pallas import tpu_sc as plsc

R = 128
N = 32768
L = 16
NC = 2
NS = 16
NW = NC * NS
RPW = R // NW
UNROLL = 8
PLU = 4
STEPS = N // (L * UNROLL)

_mesh = plsc.VectorSubcoreMesh(core_axis_name="c", subcore_axis_name="s")


@functools.partial(
    pl.kernel,
    mesh=_mesh,
    out_type=jax.ShapeDtypeStruct((NW, L), jnp.float32),
    scratch_types=[
        pltpu.VMEM((N,), jnp.float32),
        pltpu.VMEM((N,), jnp.float32),
        pltpu.VMEM((RPW, 8, 128), jnp.float32),
        pltpu.VMEM((L,), jnp.float32),
        pltpu.SemaphoreType.DMA,
        pltpu.SemaphoreType.DMA,
        pltpu.SemaphoreType.DMA,
    ],
)
def _sc_argmax_gather(m_hbm, hypo_hbm, out_hbm, buf0, buf1, gwin, obuf,
                      ldsem0, ldsem1, wsem):
    wid = lax.axis_index("s") * NC + lax.axis_index("c")
    base_row = wid * RPW
    bufs = (buf0, buf1)
    sems = (ldsem0, ldsem1)
    lane = lax.iota(jnp.int32, L)

    copies = [None, None]
    copies[0] = pltpu.async_copy(m_hbm.at[base_row], bufs[0], sems[0])

    win_copies = []
    offs = []
    for r in range(RPW):
        b = r % 2
        if r + 1 < RPW:
            nb = (r + 1) % 2
            copies[nb] = pltpu.async_copy(
                m_hbm.at[base_row + r + 1], bufs[nb], sems[nb])
        copies[b].wait()
        buf = bufs[b]

        neg_inf = jnp.full((L,), -jnp.inf, jnp.float32)
        zeros = jnp.zeros((L,), jnp.int32)

        def body(i, carry, buf=buf):
            vmaxs, vits = carry
            base = i * (L * UNROLL)
            isplat = jnp.full((L,), i, jnp.int32)
            nmax, nit = [], []
            for u in range(UNROLL):
                v = buf[pl.ds(base + u * L, L)]
                mask = v > vmaxs[u]
                nmax.append(jnp.where(mask, v, vmaxs[u]))
                nit.append(jnp.where(mask, isplat, vits[u]))
            return tuple(nmax), tuple(nit)

        vmaxs, vits = plsc.parallel_loop(
            0, STEPS, 1, unroll=PLU,
            carry=((neg_inf,) * UNROLL, (zeros,) * UNROLL))(body)

        vmax = vmaxs[0]
        vidx = vits[0] * (L * UNROLL) + lane
        for u in range(1, UNROLL):
            vidx_u = vits[u] * (L * UNROLL) + (u * L) + lane
            take = (vmaxs[u] > vmax) | ((vmaxs[u] == vmax) & (vidx_u < vidx))
            vmax = jnp.where(take, vmaxs[u], vmax)
            vidx = jnp.where(take, vidx_u, vidx)

        for d in (8, 4, 2, 1):
            perm = jnp.bitwise_xor(lane, jnp.int32(d))
            vmax2 = vmax.at[perm].get(mode="promise_in_bounds")
            vidx2 = vidx.at[perm].get(mode="promise_in_bounds")
            take = (vmax2 > vmax) | ((vmax2 == vmax) & (vidx2 < vidx))
            vmax = jnp.where(take, vmax2, vmax)
            vidx = jnp.where(take, vidx2, vidx)

        row = base_row + r
        col = vidx[0]
        row8 = pl.multiple_of(row & jnp.int32(-8), 8)
        col128 = pl.multiple_of(col & jnp.int32(-128), 128)
        offs.append((row & jnp.int32(7), col & jnp.int32(127)))
        win_copies.append(pltpu.async_copy(
            hypo_hbm.at[pl.ds(row8, 8), pl.ds(col128, 128)],
            gwin.at[r], wsem))

    for cp in win_copies:
        cp.wait()

    outvec = jnp.zeros((L,), jnp.float32)
    for r in range(RPW):
        sub, off = offs[r]
        v = gwin[r, sub, pl.ds(off & jnp.int32(-16), L)]
        wv = v.at[jnp.full((L,), off & jnp.int32(15))].get(
            mode="promise_in_bounds")
        outvec = jnp.where(lane == r, wv, outvec)
    obuf[...] = outvec
    pltpu.sync_copy(obuf, out_hbm.at[wid])


def kernel(hypo, m):
    out2d = _sc_argmax_gather(m, hypo)
    return out2d[:, :RPW].reshape(R)


# speedup vs baseline: 1.1648x; 1.0233x over previous
import functools

import jax
import jax.numpy as jnp
from jax import lax
from jax.experimental import pallas as pl
from jax.experimental.pallas import tpu as pltpu
from jax.experimental.pallas import tpu_sc as plsc

R = 128
N = 32768
L = 16
NC = 2
NS = 16
NW = NC * NS
TCR = 32
RPW = (R - TCR) // NW
UNROLL = 8
PLU = 4
STEPS = N // (L * UNROLL)

_mesh = plsc.VectorSubcoreMesh(core_axis_name="c", subcore_axis_name="s")


@functools.partial(
    pl.kernel,
    mesh=_mesh,
    out_type=jax.ShapeDtypeStruct((NW, L), jnp.float32),
    scratch_types=[
        pltpu.VMEM((N,), jnp.float32),
        pltpu.VMEM((N,), jnp.float32),
        pltpu.VMEM((RPW, 8, 128), jnp.float32),
        pltpu.VMEM((L,), jnp.float32),
        pltpu.SemaphoreType.DMA,
        pltpu.SemaphoreType.DMA,
        pltpu.SemaphoreType.DMA,
    ],
)
def _sc_argmax_gather(m_hbm, hypo_hbm, out_hbm, buf0, buf1, gwin, obuf,
                      ldsem0, ldsem1, wsem):
    wid = lax.axis_index("s") * NC + lax.axis_index("c")
    base_row = TCR + wid * RPW
    bufs = (buf0, buf1)
    sems = (ldsem0, ldsem1)
    lane = lax.iota(jnp.int32, L)

    copies = [None, None]
    copies[0] = pltpu.async_copy(m_hbm.at[base_row], bufs[0], sems[0])

    win_copies = []
    offs = []
    for r in range(RPW):
        b = r % 2
        if r + 1 < RPW:
            nb = (r + 1) % 2
            copies[nb] = pltpu.async_copy(
                m_hbm.at[base_row + r + 1], bufs[nb], sems[nb])
        copies[b].wait()
        buf = bufs[b]

        neg_inf = jnp.full((L,), -jnp.inf, jnp.float32)
        zeros = jnp.zeros((L,), jnp.int32)

        def body(i, carry, buf=buf):
            vmaxs, vits = carry
            base = i * (L * UNROLL)
            isplat = jnp.full((L,), i, jnp.int32)
            nmax, nit = [], []
            for u in range(UNROLL):
                v = buf[pl.ds(base + u * L, L)]
                mask = v > vmaxs[u]
                nmax.append(jnp.where(mask, v, vmaxs[u]))
                nit.append(jnp.where(mask, isplat, vits[u]))
            return tuple(nmax), tuple(nit)

        vmaxs, vits = plsc.parallel_loop(
            0, STEPS, 1, unroll=PLU,
            carry=((neg_inf,) * UNROLL, (zeros,) * UNROLL))(body)

        vmax = vmaxs[0]
        vidx = vits[0] * (L * UNROLL) + lane
        for u in range(1, UNROLL):
            vidx_u = vits[u] * (L * UNROLL) + (u * L) + lane
            take = (vmaxs[u] > vmax) | ((vmaxs[u] == vmax) & (vidx_u < vidx))
            vmax = jnp.where(take, vmaxs[u], vmax)
            vidx = jnp.where(take, vidx_u, vidx)

        for d in (8, 4, 2, 1):
            perm = jnp.bitwise_xor(lane, jnp.int32(d))
            vmax2 = vmax.at[perm].get(mode="promise_in_bounds")
            vidx2 = vidx.at[perm].get(mode="promise_in_bounds")
            take = (vmax2 > vmax) | ((vmax2 == vmax) & (vidx2 < vidx))
            vmax = jnp.where(take, vmax2, vmax)
            vidx = jnp.where(take, vidx2, vidx)

        row = base_row + r
        col = vidx[0]
        row8 = pl.multiple_of(row & jnp.int32(-8), 8)
        col128 = pl.multiple_of(col & jnp.int32(-128), 128)
        offs.append((row & jnp.int32(7), col & jnp.int32(127)))
        win_copies.append(pltpu.async_copy(
            hypo_hbm.at[pl.ds(row8, 8), pl.ds(col128, 128)],
            gwin.at[r], wsem))

    for cp in win_copies:
        cp.wait()

    outvec = jnp.zeros((L,), jnp.float32)
    for r in range(RPW):
        sub, off = offs[r]
        v = gwin[r, sub, pl.ds(off & jnp.int32(-16), L)]
        wv = v.at[jnp.full((L,), off & jnp.int32(15))].get(
            mode="promise_in_bounds")
        outvec = jnp.where(lane == r, wv, outvec)
    obuf[...] = outvec
    pltpu.sync_copy(obuf, out_hbm.at[wid])


def _tc_body(m_ref, h_ref, o_ref):
    x = m_ref[...]
    h = h_ref[...]
    idx = lax.broadcasted_iota(jnp.int32, (8, N), 1)
    mx = jnp.max(x, axis=1, keepdims=True)
    eq = x == mx
    big = jnp.full((8, N), jnp.int32(2**30))
    am = jnp.min(jnp.where(eq, idx, big), axis=1)
    sel = idx == am[:, None]
    o_ref[...] = jnp.sum(jnp.where(sel, h, 0.0), axis=1).reshape(1, 8, 1)


_tc_argmax_gather = pl.pallas_call(
    _tc_body,
    grid=(TCR // 8,),
    in_specs=[
        pl.BlockSpec((8, N), lambda i: (i, 0)),
        pl.BlockSpec((8, N), lambda i: (i, 0)),
    ],
    out_specs=pl.BlockSpec((1, 8, 1), lambda i: (i, 0, 0)),
    out_shape=jax.ShapeDtypeStruct((TCR // 8, 8, 1), jnp.float32),
)


def kernel(hypo, m):
    out_sc = _sc_argmax_gather(m, hypo)
    out_tc = _tc_argmax_gather(m, hypo)
    return jnp.concatenate(
        [out_tc.reshape(TCR), out_sc[:, :RPW].reshape(R - TCR)])
